# Initial kernel scaffold; baseline (speedup 1.0000x reference)
import jax
import jax.numpy as jnp
from jax.experimental import pallas as pl


def kernel(x, edge_index, W_self_0, W_neigh_0, b_0, W_self_1, W_neigh_1, b_1, W_self_2, W_neigh_2, b_2):
    raise NotImplementedError("write your pallas kernel here")


# speedup vs baseline: 3.5260x; 3.5260x over previous
import functools

import jax
import jax.numpy as jnp
from jax import lax
from jax.experimental import pallas as pl
from jax.experimental.pallas import tpu as pltpu
from jax.experimental.pallas import tpu_sc as plsc

N = 10000
D = 128
NC, NS = 2, 16
NW = NC * NS
CH = 128
N_ACC = 10240
ROWS_PER_TILE = N_ACC // NS
NBUF = 2
NCHUNK = 80
EPW = NCHUNK * CH
E_PAD = NW * EPW
NGROUP = NCHUNK // NBUF


def _sc_agg_body(h_hbm, src_hbm, dst_hbm, out_hbm,
                 sidx_v, didx_v, rows_v, zrow_v, acc_s,
                 gsem0, gsem1, isem0, isem1):
    gsems = (gsem0, gsem1)
    isems = (isem0, isem1)
    c = lax.axis_index("c")
    s = lax.axis_index("s")
    wid = s * NC + c
    base = wid * EPW

    for i in range(16):
        for j in range(D // 16):
            zrow_v[i, pl.ds(j * 16, 16)] = jnp.zeros((16,), jnp.float32)
    zoff = s * ROWS_PER_TILE
    for k in range(ROWS_PER_TILE // 16):
        pltpu.sync_copy(zrow_v, acc_s.at[pl.ds(zoff + k * 16, 16)])
    pltpu.sync_copy(src_hbm.at[pl.ds(base, EPW)], sidx_v)
    plsc.subcore_barrier()

    def gstart(b, j):
        pltpu.make_async_copy(h_hbm.at[sidx_v.at[pl.ds(j * CH, CH)]],
                              rows_v.at[b], gsems[b]).start()

    def gwait(b):
        pltpu.make_async_copy(h_hbm.at[sidx_v.at[pl.ds(0, CH)]],
                              rows_v.at[b], gsems[b]).wait()

    def istart(b, j):
        pltpu.make_async_copy(dst_hbm.at[pl.ds(base + j * CH, CH)],
                              didx_v.at[b], isems[b]).start()

    def iwait(b):
        pltpu.make_async_copy(dst_hbm.at[pl.ds(base, CH)],
                              didx_v.at[b], isems[b]).wait()

    def scatter(b):
        pltpu.sync_copy(rows_v.at[b], acc_s.at[didx_v.at[b]], add=True)

    for b in range(NBUF):
        istart(b, b)
        gstart(b, b)

    def group(g, carry):
        j0 = g * NBUF
        for b in range(NBUF):
            gwait(b)
            iwait(b)
            scatter(b)
            istart(b, j0 + b + NBUF)
            gstart(b, j0 + b + NBUF)
        return carry

    lax.fori_loop(0, NGROUP - 1, group, 0)
    for b in range(NBUF):
        gwait(b)
        iwait(b)
        scatter(b)
    plsc.subcore_barrier()

    pltpu.sync_copy(acc_s.at[pl.ds(zoff, ROWS_PER_TILE)],
                    out_hbm.at[c, pl.ds(zoff, ROWS_PER_TILE)])


def _make_sc_agg():
    mesh = plsc.VectorSubcoreMesh(core_axis_name="c", subcore_axis_name="s",
                                  num_cores=NC, num_subcores=NS)
    out_type = jax.ShapeDtypeStruct((NC, N_ACC, D), jnp.float32)
    scratch = [
        pltpu.VMEM((EPW,), jnp.int32),
        pltpu.VMEM((NBUF, CH), jnp.int32),
        pltpu.VMEM((NBUF, CH, D), jnp.float32),
        pltpu.VMEM((16, D), jnp.float32),
        pltpu.VMEM_SHARED((N_ACC, D), jnp.float32),
        pltpu.SemaphoreType.DMA,
        pltpu.SemaphoreType.DMA,
        pltpu.SemaphoreType.DMA,
        pltpu.SemaphoreType.DMA,
    ]
    return pl.kernel(_sc_agg_body, out_type=out_type, mesh=mesh,
                     scratch_types=scratch)


def _sc_deg_body(dst_hbm, deg_hbm, didx_v, ones_v, zrow_v, deg_s):
    c = lax.axis_index("c")
    s = lax.axis_index("s")
    wid = s * NC + c
    base = wid * EPW
    zoff = s * ROWS_PER_TILE

    for i in range(16):
        for j in range(D // 16):
            zrow_v[i, pl.ds(j * 16, 16)] = jnp.zeros((16,), jnp.float32)
    for i in range(CH):
        for j in range(D // 16):
            ones_v[i, pl.ds(j * 16, 16)] = jnp.ones((16,), jnp.float32)
    for k in range(ROWS_PER_TILE // 16):
        pltpu.sync_copy(zrow_v, deg_s.at[pl.ds(zoff + k * 16, 16)])
    pltpu.sync_copy(dst_hbm.at[pl.ds(base, EPW)], didx_v)
    plsc.subcore_barrier()

    def chunk(j, carry):
        pltpu.sync_copy(ones_v, deg_s.at[didx_v.at[pl.ds(j * CH, CH)]],
                        add=True)
        return carry

    lax.fori_loop(0, NCHUNK, chunk, 0)
    plsc.subcore_barrier()
    pltpu.sync_copy(deg_s.at[pl.ds(zoff, ROWS_PER_TILE)],
                    deg_hbm.at[c, pl.ds(zoff, ROWS_PER_TILE)])


def _make_sc_deg():
    mesh = plsc.VectorSubcoreMesh(core_axis_name="c", subcore_axis_name="s",
                                  num_cores=NC, num_subcores=NS)
    out_type = jax.ShapeDtypeStruct((NC, N_ACC, D), jnp.float32)
    scratch = [
        pltpu.VMEM((EPW,), jnp.int32),
        pltpu.VMEM((CH, D), jnp.float32),
        pltpu.VMEM((16, D), jnp.float32),
        pltpu.VMEM_SHARED((N_ACC, D), jnp.float32),
    ]
    return pl.kernel(_sc_deg_body, out_type=out_type, mesh=mesh,
                     scratch_types=scratch)


def _dense_body(act, h_ref, p_ref, dg_ref, ws_ref, wn_ref, b_ref, o_ref):
    h = h_ref[...]
    psum = p_ref[0] + p_ref[1]
    deg = dg_ref[0] + dg_ref[1]
    inv = 1.0 / jnp.maximum(deg, 1.0)
    agg = psum * inv
    out = (jnp.dot(h, ws_ref[...], preferred_element_type=jnp.float32)
           + jnp.dot(agg, wn_ref[...], preferred_element_type=jnp.float32)
           + b_ref[...])
    if act:
        out = jnp.maximum(out, 0.0)
    o_ref[...] = out


def _make_dense(act):
    bn = 1024
    grid = N_ACC // bn
    return pl.pallas_call(
        functools.partial(_dense_body, act),
        grid=(grid,),
        in_specs=[
            pl.BlockSpec((bn, D), lambda i: (i, 0)),
            pl.BlockSpec((NC, bn, D), lambda i: (0, i, 0)),
            pl.BlockSpec((NC, bn, 1), lambda i: (0, i, 0)),
            pl.BlockSpec((D, D), lambda i: (0, 0)),
            pl.BlockSpec((D, D), lambda i: (0, 0)),
            pl.BlockSpec((1, D), lambda i: (0, 0)),
        ],
        out_specs=pl.BlockSpec((bn, D), lambda i: (i, 0)),
        out_shape=jax.ShapeDtypeStruct((N_ACC, D), jnp.float32),
    )


@functools.lru_cache(maxsize=None)
def _kernels():
    return (_make_sc_agg(), _make_sc_deg(),
            _make_dense(True), _make_dense(False))


def kernel(x, edge_index, W_self_0, W_neigh_0, b_0, W_self_1, W_neigh_1,
           b_1, W_self_2, W_neigh_2, b_2):
    _sc_agg, _sc_deg, _dense_relu, _dense_lin = _kernels()
    src = edge_index[0]
    dst = edge_index[1]
    pad = E_PAD - src.shape[0]
    src_p = jnp.concatenate([src, jnp.zeros((pad,), jnp.int32)])
    dst_p = jnp.concatenate([dst, jnp.full((pad,), N, jnp.int32)])
    h = jnp.pad(x, ((0, N_ACC - N), (0, 0)))

    deg = _sc_deg(dst_p)[:, :, 0:1]
    p = _sc_agg(h, src_p, dst_p)
    h = _dense_relu(h, p, deg, W_self_0, W_neigh_0, b_0.reshape(1, D))
    p = _sc_agg(h, src_p, dst_p)
    h = _dense_relu(h, p, deg, W_self_1, W_neigh_1, b_1.reshape(1, D))
    p = _sc_agg(h, src_p, dst_p)
    h = _dense_lin(h, p, deg, W_self_2, W_neigh_2, b_2.reshape(1, D))
    return h[:N]


# speedup vs baseline: 3.6397x; 1.0322x over previous
import functools

import jax
import jax.numpy as jnp
from jax import lax
from jax.experimental import pallas as pl
from jax.experimental.pallas import tpu as pltpu
from jax.experimental.pallas import tpu_sc as plsc

N = 10000
D = 128
NC, NS = 2, 16
NW = NC * NS
CH = 128
N_ACC = 10240
ROWS_PER_TILE = N_ACC // NS
NBUF = 2
NCHUNK = 80
EPW = NCHUNK * CH
E_PAD = NW * EPW
NGROUP = NCHUNK // NBUF
PAIR_CHUNKS = 2 * NCHUNK
NCH0 = 112
NCH1 = PAIR_CHUNKS - NCH0
E_ALLOC = E_PAD + (NCH0 - NCH1) * CH


def _sc_agg_body(h_hbm, src_hbm, dst_hbm, out_hbm,
                 sidx_v, didx_v, rows_v, zrow_v, acc_s,
                 gsem0, gsem1, isem0, isem1):
    gsems = (gsem0, gsem1)
    isems = (isem0, isem1)
    c = lax.axis_index("c")
    s = lax.axis_index("s")
    base = s * (PAIR_CHUNKS * CH) + jnp.where(c == 0, 0, NCH0 * CH)
    nchunk = jnp.where(c == 0, NCH0, NCH1)

    for i in range(8):
        for j in range(D // 16):
            zrow_v[i, pl.ds(j * 16, 16)] = jnp.zeros((16,), jnp.float32)
    zoff = s * ROWS_PER_TILE
    for k in range(ROWS_PER_TILE // 8):
        pltpu.sync_copy(zrow_v, acc_s.at[pl.ds(zoff + k * 8, 8)])
    pltpu.sync_copy(src_hbm.at[pl.ds(base, NCH0 * CH)], sidx_v)
    plsc.subcore_barrier()

    def gstart(b, j):
        pltpu.make_async_copy(h_hbm.at[sidx_v.at[pl.ds(j * CH, CH)]],
                              rows_v.at[b], gsems[b]).start()

    def gwait(b):
        pltpu.make_async_copy(h_hbm.at[sidx_v.at[pl.ds(0, CH)]],
                              rows_v.at[b], gsems[b]).wait()

    def istart(b, j):
        pltpu.make_async_copy(dst_hbm.at[pl.ds(base + j * CH, CH)],
                              didx_v.at[b], isems[b]).start()

    def iwait(b):
        pltpu.make_async_copy(dst_hbm.at[pl.ds(base, CH)],
                              didx_v.at[b], isems[b]).wait()

    def scatter(b):
        pltpu.sync_copy(rows_v.at[b], acc_s.at[didx_v.at[b]], add=True)

    for b in range(NBUF):
        istart(b, b)
        gstart(b, b)

    def group(g, carry):
        j0 = g * NBUF
        for b in range(NBUF):
            gwait(b)
            iwait(b)
            scatter(b)
            istart(b, j0 + b + NBUF)
            gstart(b, j0 + b + NBUF)
        return carry

    lax.fori_loop(0, nchunk // NBUF - 1, group, 0)
    for b in range(NBUF):
        gwait(b)
        iwait(b)
        scatter(b)
    plsc.subcore_barrier()

    pltpu.sync_copy(acc_s.at[pl.ds(zoff, ROWS_PER_TILE)],
                    out_hbm.at[c, pl.ds(zoff, ROWS_PER_TILE)])


def _make_sc_agg():
    mesh = plsc.VectorSubcoreMesh(core_axis_name="c", subcore_axis_name="s",
                                  num_cores=NC, num_subcores=NS)
    out_type = jax.ShapeDtypeStruct((NC, N_ACC, D), jnp.float32)
    scratch = [
        pltpu.VMEM((NCH0 * CH,), jnp.int32),
        pltpu.VMEM((NBUF, CH), jnp.int32),
        pltpu.VMEM((NBUF, CH, D), jnp.float32),
        pltpu.VMEM((8, D), jnp.float32),
        pltpu.VMEM_SHARED((N_ACC, D), jnp.float32),
        pltpu.SemaphoreType.DMA,
        pltpu.SemaphoreType.DMA,
        pltpu.SemaphoreType.DMA,
        pltpu.SemaphoreType.DMA,
    ]
    return pl.kernel(_sc_agg_body, out_type=out_type, mesh=mesh,
                     scratch_types=scratch)


def _sc_deg_body(dst_hbm, deg_hbm, didx_v, ones_v, zrow_v, deg_s):
    c = lax.axis_index("c")
    s = lax.axis_index("s")
    wid = s * NC + c
    base = wid * EPW
    zoff = s * ROWS_PER_TILE

    for i in range(16):
        for j in range(D // 16):
            zrow_v[i, pl.ds(j * 16, 16)] = jnp.zeros((16,), jnp.float32)
    for i in range(CH):
        for j in range(D // 16):
            ones_v[i, pl.ds(j * 16, 16)] = jnp.ones((16,), jnp.float32)
    for k in range(ROWS_PER_TILE // 16):
        pltpu.sync_copy(zrow_v, deg_s.at[pl.ds(zoff + k * 16, 16)])
    pltpu.sync_copy(dst_hbm.at[pl.ds(base, EPW)], didx_v)
    plsc.subcore_barrier()

    def chunk(j, carry):
        pltpu.sync_copy(ones_v, deg_s.at[didx_v.at[pl.ds(j * CH, CH)]],
                        add=True)
        return carry

    lax.fori_loop(0, NCHUNK, chunk, 0)
    plsc.subcore_barrier()
    pltpu.sync_copy(deg_s.at[pl.ds(zoff, ROWS_PER_TILE)],
                    deg_hbm.at[c, pl.ds(zoff, ROWS_PER_TILE)])


def _make_sc_deg():
    mesh = plsc.VectorSubcoreMesh(core_axis_name="c", subcore_axis_name="s",
                                  num_cores=NC, num_subcores=NS)
    out_type = jax.ShapeDtypeStruct((NC, N_ACC, D), jnp.float32)
    scratch = [
        pltpu.VMEM((EPW,), jnp.int32),
        pltpu.VMEM((CH, D), jnp.float32),
        pltpu.VMEM((16, D), jnp.float32),
        pltpu.VMEM_SHARED((N_ACC, D), jnp.float32),
    ]
    return pl.kernel(_sc_deg_body, out_type=out_type, mesh=mesh,
                     scratch_types=scratch)


def _dense_body(act, h_ref, p_ref, dg_ref, ws_ref, wn_ref, b_ref, o_ref):
    h = h_ref[...]
    psum = p_ref[0] + p_ref[1]
    deg = dg_ref[0] + dg_ref[1]
    inv = 1.0 / jnp.maximum(deg, 1.0)
    agg = psum * inv
    out = (jnp.dot(h, ws_ref[...], preferred_element_type=jnp.float32)
           + jnp.dot(agg, wn_ref[...], preferred_element_type=jnp.float32)
           + b_ref[...])
    if act:
        out = jnp.maximum(out, 0.0)
    o_ref[...] = out


def _make_dense(act):
    bn = 1024
    grid = N_ACC // bn
    return pl.pallas_call(
        functools.partial(_dense_body, act),
        grid=(grid,),
        in_specs=[
            pl.BlockSpec((bn, D), lambda i: (i, 0)),
            pl.BlockSpec((NC, bn, D), lambda i: (0, i, 0)),
            pl.BlockSpec((NC, bn, 1), lambda i: (0, i, 0)),
            pl.BlockSpec((D, D), lambda i: (0, 0)),
            pl.BlockSpec((D, D), lambda i: (0, 0)),
            pl.BlockSpec((1, D), lambda i: (0, 0)),
        ],
        out_specs=pl.BlockSpec((bn, D), lambda i: (i, 0)),
        out_shape=jax.ShapeDtypeStruct((N_ACC, D), jnp.float32),
    )


@functools.lru_cache(maxsize=None)
def _kernels():
    return (_make_sc_agg(), _make_sc_deg(),
            _make_dense(True), _make_dense(False))


def kernel(x, edge_index, W_self_0, W_neigh_0, b_0, W_self_1, W_neigh_1,
           b_1, W_self_2, W_neigh_2, b_2):
    _sc_agg, _sc_deg, _dense_relu, _dense_lin = _kernels()
    src = edge_index[0]
    dst = edge_index[1]
    pad = E_ALLOC - src.shape[0]
    src_p = jnp.concatenate([src, jnp.zeros((pad,), jnp.int32)])
    dst_p = jnp.concatenate([dst, jnp.full((pad,), N, jnp.int32)])
    h = jnp.pad(x, ((0, N_ACC - N), (0, 0)))

    deg = _sc_deg(dst_p)[:, :, 0:1]
    p = _sc_agg(h, src_p, dst_p)
    h = _dense_relu(h, p, deg, W_self_0, W_neigh_0, b_0.reshape(1, D))
    p = _sc_agg(h, src_p, dst_p)
    h = _dense_relu(h, p, deg, W_self_1, W_neigh_1, b_1.reshape(1, D))
    p = _sc_agg(h, src_p, dst_p)
    h = _dense_lin(h, p, deg, W_self_2, W_neigh_2, b_2.reshape(1, D))
    return h[:N]


# speedup vs baseline: 3.7149x; 1.0207x over previous
import functools

import jax
import jax.numpy as jnp
from jax import lax
from jax.experimental import pallas as pl
from jax.experimental.pallas import tpu as pltpu
from jax.experimental.pallas import tpu_sc as plsc

N = 10000
D = 128
NC, NS = 2, 16
NW = NC * NS
CH = 128
N_ACC = 10240
ROWS_PER_TILE = N_ACC // NS
NBUF = 2
NIBUF = 4
NCHUNK = 80
EPW = NCHUNK * CH
E_PAD = NW * EPW
E_ALLOC = E_PAD
ZR = 64
PAIR_CHUNKS = 2 * NCHUNK
NCH0 = 132
NCH1 = PAIR_CHUNKS - NCH0


def _sc_agg_body(h_hbm, src_hbm, dst_hbm, out_hbm,
                 sidx_v, didx_v, rows_v, zrow_v, acc_s,
                 gsem0, gsem1, ssem0, ssem1, ssem2, ssem3,
                 dsem0, dsem1, dsem2, dsem3, zsem):
    gsems = (gsem0, gsem1)
    ssems = (ssem0, ssem1, ssem2, ssem3)
    dsems = (dsem0, dsem1, dsem2, dsem3)
    c = lax.axis_index("c")
    s = lax.axis_index("s")
    base = s * (PAIR_CHUNKS * CH) + jnp.where(c == 0, 0, NCH0 * CH)
    nchunk = jnp.where(c == 0, NCH0, NCH1)
    zoff = s * ROWS_PER_TILE

    def istart(k, j):
        pltpu.make_async_copy(src_hbm.at[pl.ds(base + j * CH, CH)],
                              sidx_v.at[k], ssems[k]).start()
        pltpu.make_async_copy(dst_hbm.at[pl.ds(base + j * CH, CH)],
                              didx_v.at[k], dsems[k]).start()

    def swait(k):
        pltpu.make_async_copy(src_hbm.at[pl.ds(base, CH)],
                              sidx_v.at[k], ssems[k]).wait()

    def dwait(k):
        pltpu.make_async_copy(dst_hbm.at[pl.ds(base, CH)],
                              didx_v.at[k], dsems[k]).wait()

    def gstart(b, k):
        pltpu.make_async_copy(h_hbm.at[sidx_v.at[k]],
                              rows_v.at[b], gsems[b]).start()

    def gwait(b):
        pltpu.make_async_copy(h_hbm.at[sidx_v.at[0]],
                              rows_v.at[b], gsems[b]).wait()

    def scatter(b, k):
        pltpu.sync_copy(rows_v.at[b], acc_s.at[didx_v.at[k]], add=True)

    for i in range(ZR):
        for j in range(D // 16):
            zrow_v[i, pl.ds(j * 16, 16)] = jnp.zeros((16,), jnp.float32)
    for k in range(ROWS_PER_TILE // ZR):
        pltpu.make_async_copy(zrow_v, acc_s.at[pl.ds(zoff + k * ZR, ZR)],
                              zsem).start()
    for k in range(NIBUF):
        istart(k, k)
    for b in range(NBUF):
        swait(b)
        gstart(b, b)
    for k in range(ROWS_PER_TILE // ZR):
        pltpu.make_async_copy(zrow_v, acc_s.at[pl.ds(zoff + k * ZR, ZR)],
                              zsem).wait()
    plsc.subcore_barrier()

    def group(g, carry):
        j0 = g * NIBUF
        for b in range(NIBUF):
            gwait(b % NBUF)
            dwait(b)
            scatter(b % NBUF, b)
            istart(b, j0 + b + NIBUF)
            swait((b + NBUF) % NIBUF)
            gstart(b % NBUF, (b + NBUF) % NIBUF)
        return carry

    lax.fori_loop(0, (nchunk - NIBUF) // NIBUF, group, 0)
    for b in range(NBUF):
        gwait(b % NBUF)
        dwait(b)
        scatter(b % NBUF, b)
        swait((b + NBUF) % NIBUF)
        gstart(b % NBUF, (b + NBUF) % NIBUF)
    for b in range(NBUF, NIBUF):
        gwait(b % NBUF)
        dwait(b)
        scatter(b % NBUF, b)
    plsc.subcore_barrier()

    pltpu.sync_copy(acc_s.at[pl.ds(zoff, ROWS_PER_TILE)],
                    out_hbm.at[c, pl.ds(zoff, ROWS_PER_TILE)])


def _make_sc_agg():
    mesh = plsc.VectorSubcoreMesh(core_axis_name="c", subcore_axis_name="s",
                                  num_cores=NC, num_subcores=NS)
    out_type = jax.ShapeDtypeStruct((NC, N_ACC, D), jnp.float32)
    scratch = [
        pltpu.VMEM((NIBUF, CH), jnp.int32),
        pltpu.VMEM((NIBUF, CH), jnp.int32),
        pltpu.VMEM((NBUF, CH, D), jnp.float32),
        pltpu.VMEM((ZR, D), jnp.float32),
        pltpu.VMEM_SHARED((N_ACC, D), jnp.float32),
    ] + [pltpu.SemaphoreType.DMA] * 11
    return pl.kernel(_sc_agg_body, out_type=out_type, mesh=mesh,
                     scratch_types=scratch)


def _sc_deg_body(dst_hbm, deg_hbm, didx_v, ones_v, zrow_v, deg_s, zsem):
    c = lax.axis_index("c")
    s = lax.axis_index("s")
    wid = s * NC + c
    base = wid * EPW
    zoff = s * ROWS_PER_TILE

    for i in range(ZR):
        for j in range(D // 16):
            zrow_v[i, pl.ds(j * 16, 16)] = jnp.zeros((16,), jnp.float32)
    for i in range(CH):
        for j in range(D // 16):
            ones_v[i, pl.ds(j * 16, 16)] = jnp.ones((16,), jnp.float32)
    for k in range(ROWS_PER_TILE // ZR):
        pltpu.make_async_copy(zrow_v, deg_s.at[pl.ds(zoff + k * ZR, ZR)],
                              zsem).start()
    pltpu.sync_copy(dst_hbm.at[pl.ds(base, EPW)], didx_v)
    for k in range(ROWS_PER_TILE // ZR):
        pltpu.make_async_copy(zrow_v, deg_s.at[pl.ds(zoff + k * ZR, ZR)],
                              zsem).wait()
    plsc.subcore_barrier()

    def chunk(j, carry):
        pltpu.sync_copy(ones_v, deg_s.at[didx_v.at[pl.ds(j * CH, CH)]],
                        add=True)
        return carry

    lax.fori_loop(0, NCHUNK, chunk, 0)
    plsc.subcore_barrier()
    pltpu.sync_copy(deg_s.at[pl.ds(zoff, ROWS_PER_TILE)],
                    deg_hbm.at[c, pl.ds(zoff, ROWS_PER_TILE)])


def _make_sc_deg():
    mesh = plsc.VectorSubcoreMesh(core_axis_name="c", subcore_axis_name="s",
                                  num_cores=NC, num_subcores=NS)
    out_type = jax.ShapeDtypeStruct((NC, N_ACC, D), jnp.float32)
    scratch = [
        pltpu.VMEM((EPW,), jnp.int32),
        pltpu.VMEM((CH, D), jnp.float32),
        pltpu.VMEM((ZR, D), jnp.float32),
        pltpu.VMEM_SHARED((N_ACC, D), jnp.float32),
        pltpu.SemaphoreType.DMA,
    ]
    return pl.kernel(_sc_deg_body, out_type=out_type, mesh=mesh,
                     scratch_types=scratch)


def _dense_body(act, h_ref, p_ref, dg_ref, ws_ref, wn_ref, b_ref, o_ref):
    h = h_ref[...]
    psum = p_ref[0] + p_ref[1]
    deg = dg_ref[0] + dg_ref[1]
    inv = 1.0 / jnp.maximum(deg, 1.0)
    agg = psum * inv
    out = (jnp.dot(h, ws_ref[...], preferred_element_type=jnp.float32)
           + jnp.dot(agg, wn_ref[...], preferred_element_type=jnp.float32)
           + b_ref[...])
    if act:
        out = jnp.maximum(out, 0.0)
    o_ref[...] = out


def _make_dense(act):
    bn = 1024
    grid = N_ACC // bn
    return pl.pallas_call(
        functools.partial(_dense_body, act),
        grid=(grid,),
        in_specs=[
            pl.BlockSpec((bn, D), lambda i: (i, 0)),
            pl.BlockSpec((NC, bn, D), lambda i: (0, i, 0)),
            pl.BlockSpec((NC, bn, 1), lambda i: (0, i, 0)),
            pl.BlockSpec((D, D), lambda i: (0, 0)),
            pl.BlockSpec((D, D), lambda i: (0, 0)),
            pl.BlockSpec((1, D), lambda i: (0, 0)),
        ],
        out_specs=pl.BlockSpec((bn, D), lambda i: (i, 0)),
        out_shape=jax.ShapeDtypeStruct((N_ACC, D), jnp.float32),
    )


@functools.lru_cache(maxsize=None)
def _kernels():
    return (_make_sc_agg(), _make_sc_deg(),
            _make_dense(True), _make_dense(False))


def kernel(x, edge_index, W_self_0, W_neigh_0, b_0, W_self_1, W_neigh_1,
           b_1, W_self_2, W_neigh_2, b_2):
    _sc_agg, _sc_deg, _dense_relu, _dense_lin = _kernels()
    src = edge_index[0]
    dst = edge_index[1]
    pad = E_ALLOC - src.shape[0]
    src_p = jnp.concatenate([src, jnp.zeros((pad,), jnp.int32)])
    dst_p = jnp.concatenate([dst, jnp.full((pad,), N, jnp.int32)])
    h = jnp.pad(x, ((0, N_ACC - N), (0, 0)))

    deg = _sc_deg(dst_p)[:, :, 0:1]
    p = _sc_agg(h, src_p, dst_p)
    h = _dense_relu(h, p, deg, W_self_0, W_neigh_0, b_0.reshape(1, D))
    p = _sc_agg(h, src_p, dst_p)
    h = _dense_relu(h, p, deg, W_self_1, W_neigh_1, b_1.reshape(1, D))
    p = _sc_agg(h, src_p, dst_p)
    h = _dense_lin(h, p, deg, W_self_2, W_neigh_2, b_2.reshape(1, D))
    return h[:N]


# speedup vs baseline: 10.5709x; 2.8455x over previous
import functools

import jax
import jax.numpy as jnp
from jax import lax
from jax.experimental import pallas as pl
from jax.experimental.pallas import tpu as pltpu
from jax.experimental.pallas import tpu_sc as plsc

N = 10000
D = 128
NC, NS = 2, 16
NW = NC * NS
CH = 128
N_ACC = 10240
ROWS_PER_TILE = N_ACC // NS
NBUF = 2
NIBUF = 4
NCHUNK = 80
EPW = NCHUNK * CH
E_PAD = NW * EPW
E_ALLOC = E_PAD
ZR = 64
PAIR_CHUNKS = 2 * NCHUNK
NCH0 = 80
NCH1 = PAIR_CHUNKS - NCH0


def _sc_agg_body(h_hbm, src_hbm, dst_hbm, out_hbm,
                 sidx_v, didx_v, rows_v, zrow_v, acc_s,
                 gsem0, gsem1, ssem0, ssem1, ssem2, ssem3,
                 dsem0, dsem1, dsem2, dsem3, zsem):
    gsems = (gsem0, gsem1)
    ssems = (ssem0, ssem1, ssem2, ssem3)
    dsems = (dsem0, dsem1, dsem2, dsem3)
    c = lax.axis_index("c")
    s = lax.axis_index("s")
    base = s * (PAIR_CHUNKS * CH) + jnp.where(c == 0, 0, NCH0 * CH)
    nchunk = jnp.where(c == 0, NCH0, NCH1)
    zoff = s * ROWS_PER_TILE

    def istart(k, j):
        pltpu.make_async_copy(src_hbm.at[pl.ds(base + j * CH, CH)],
                              sidx_v.at[k], ssems[k]).start()
        pltpu.make_async_copy(dst_hbm.at[pl.ds(base + j * CH, CH)],
                              didx_v.at[k], dsems[k]).start()

    def swait(k):
        pltpu.make_async_copy(src_hbm.at[pl.ds(base, CH)],
                              sidx_v.at[k], ssems[k]).wait()

    def dwait(k):
        pltpu.make_async_copy(dst_hbm.at[pl.ds(base, CH)],
                              didx_v.at[k], dsems[k]).wait()

    def gstart(b, k):
        pltpu.make_async_copy(h_hbm.at[sidx_v.at[k]],
                              rows_v.at[b], gsems[b]).start()

    def gwait(b):
        pltpu.make_async_copy(h_hbm.at[sidx_v.at[0]],
                              rows_v.at[b], gsems[b]).wait()

    def scatter(b, k):
        pltpu.sync_copy(rows_v.at[b], acc_s.at[didx_v.at[k]], add=True)

    for i in range(ZR):
        for j in range(D // 16):
            zrow_v[i, pl.ds(j * 16, 16)] = jnp.zeros((16,), jnp.float32)
    for k in range(ROWS_PER_TILE // ZR):
        pltpu.make_async_copy(zrow_v, acc_s.at[pl.ds(zoff + k * ZR, ZR)],
                              zsem).start()
    for k in range(NIBUF):
        istart(k, k)
    for b in range(NBUF):
        swait(b)
        gstart(b, b)
    for k in range(ROWS_PER_TILE // ZR):
        pltpu.make_async_copy(zrow_v, acc_s.at[pl.ds(zoff + k * ZR, ZR)],
                              zsem).wait()
    plsc.subcore_barrier()

    def group(g, carry):
        j0 = g * NIBUF
        for b in range(NIBUF):
            gwait(b % NBUF)
            dwait(b)
            scatter(b % NBUF, b)
            istart(b, j0 + b + NIBUF)
            swait((b + NBUF) % NIBUF)
            gstart(b % NBUF, (b + NBUF) % NIBUF)
        return carry

    lax.fori_loop(0, (nchunk - NIBUF) // NIBUF, group, 0)
    for b in range(NBUF):
        gwait(b % NBUF)
        dwait(b)
        scatter(b % NBUF, b)
        swait((b + NBUF) % NIBUF)
        gstart(b % NBUF, (b + NBUF) % NIBUF)
    for b in range(NBUF, NIBUF):
        gwait(b % NBUF)
        dwait(b)
        scatter(b % NBUF, b)
    plsc.subcore_barrier()

    pltpu.sync_copy(acc_s.at[pl.ds(zoff, ROWS_PER_TILE)],
                    out_hbm.at[c, pl.ds(zoff, ROWS_PER_TILE)])


def _make_sc_agg():
    mesh = plsc.VectorSubcoreMesh(core_axis_name="c", subcore_axis_name="s",
                                  num_cores=NC, num_subcores=NS)
    out_type = jax.ShapeDtypeStruct((NC, N_ACC, D), jnp.float32)
    scratch = [
        pltpu.VMEM((NIBUF, CH), jnp.int32),
        pltpu.VMEM((NIBUF, CH), jnp.int32),
        pltpu.VMEM((NBUF, CH, D), jnp.float32),
        pltpu.VMEM((ZR, D), jnp.float32),
        pltpu.VMEM_SHARED((N_ACC, D), jnp.float32),
    ] + [pltpu.SemaphoreType.DMA] * 11
    return pl.kernel(_sc_agg_body, out_type=out_type, mesh=mesh,
                     scratch_types=scratch)


def _sc_deg_body(dst_hbm, deg_hbm, didx_v, ones_v, zrow_v, deg_s, zsem):
    c = lax.axis_index("c")
    s = lax.axis_index("s")
    wid = s * NC + c
    base = wid * EPW
    zoff = s * ROWS_PER_TILE

    for i in range(ZR):
        for j in range(D // 16):
            zrow_v[i, pl.ds(j * 16, 16)] = jnp.zeros((16,), jnp.float32)
    for i in range(CH):
        for j in range(D // 16):
            ones_v[i, pl.ds(j * 16, 16)] = jnp.ones((16,), jnp.float32)
    for k in range(ROWS_PER_TILE // ZR):
        pltpu.make_async_copy(zrow_v, deg_s.at[pl.ds(zoff + k * ZR, ZR)],
                              zsem).start()
    pltpu.sync_copy(dst_hbm.at[pl.ds(base, EPW)], didx_v)
    for k in range(ROWS_PER_TILE // ZR):
        pltpu.make_async_copy(zrow_v, deg_s.at[pl.ds(zoff + k * ZR, ZR)],
                              zsem).wait()
    plsc.subcore_barrier()

    def chunk(j, carry):
        pltpu.sync_copy(ones_v, deg_s.at[didx_v.at[pl.ds(j * CH, CH)]],
                        add=True)
        return carry

    lax.fori_loop(0, NCHUNK, chunk, 0)
    plsc.subcore_barrier()
    pltpu.sync_copy(deg_s.at[pl.ds(zoff, ROWS_PER_TILE)],
                    deg_hbm.at[c, pl.ds(zoff, ROWS_PER_TILE)])


def _make_sc_deg():
    mesh = plsc.VectorSubcoreMesh(core_axis_name="c", subcore_axis_name="s",
                                  num_cores=NC, num_subcores=NS)
    out_type = jax.ShapeDtypeStruct((NC, N_ACC, D), jnp.float32)
    scratch = [
        pltpu.VMEM((EPW,), jnp.int32),
        pltpu.VMEM((CH, D), jnp.float32),
        pltpu.VMEM((ZR, D), jnp.float32),
        pltpu.VMEM_SHARED((N_ACC, D), jnp.float32),
        pltpu.SemaphoreType.DMA,
    ]
    return pl.kernel(_sc_deg_body, out_type=out_type, mesh=mesh,
                     scratch_types=scratch)


def _dense_body(act, h_ref, p_ref, dg_ref, ws_ref, wn_ref, b_ref, o_ref):
    h = h_ref[...]
    psum = p_ref[0] + p_ref[1]
    deg = dg_ref[0] + dg_ref[1]
    inv = 1.0 / jnp.maximum(deg, 1.0)
    agg = psum * inv
    out = (jnp.dot(h, ws_ref[...], preferred_element_type=jnp.float32)
           + jnp.dot(agg, wn_ref[...], preferred_element_type=jnp.float32)
           + b_ref[...])
    if act:
        out = jnp.maximum(out, 0.0)
    o_ref[...] = out


def _make_dense(act):
    bn = 1024
    grid = N_ACC // bn
    return pl.pallas_call(
        functools.partial(_dense_body, act),
        grid=(grid,),
        in_specs=[
            pl.BlockSpec((bn, D), lambda i: (i, 0)),
            pl.BlockSpec((NC, bn, D), lambda i: (0, i, 0)),
            pl.BlockSpec((NC, bn, 1), lambda i: (0, i, 0)),
            pl.BlockSpec((D, D), lambda i: (0, 0)),
            pl.BlockSpec((D, D), lambda i: (0, 0)),
            pl.BlockSpec((1, D), lambda i: (0, 0)),
        ],
        out_specs=pl.BlockSpec((bn, D), lambda i: (i, 0)),
        out_shape=jax.ShapeDtypeStruct((N_ACC, D), jnp.float32),
    )


@functools.lru_cache(maxsize=None)
def _kernels():
    return (_make_sc_agg(), _make_sc_deg(),
            _make_dense(True), _make_dense(False))


def kernel(x, edge_index, W_self_0, W_neigh_0, b_0, W_self_1, W_neigh_1,
           b_1, W_self_2, W_neigh_2, b_2):
    _sc_agg, _sc_deg, _dense_relu, _dense_lin = _kernels()
    src = edge_index[0]
    dst = edge_index[1]
    pad = E_ALLOC - src.shape[0]
    src_p = jnp.concatenate([src, jnp.arange(pad, dtype=jnp.int32) % N])
    dst_p = jnp.concatenate([dst, jnp.full((pad,), N, jnp.int32)])
    h = jnp.pad(x, ((0, N_ACC - N), (0, 0)))

    deg = _sc_deg(dst_p)[:, :, 0:1]
    p = _sc_agg(h, src_p, dst_p)
    h = _dense_relu(h, p, deg, W_self_0, W_neigh_0, b_0.reshape(1, D))
    p = _sc_agg(h, src_p, dst_p)
    h = _dense_relu(h, p, deg, W_self_1, W_neigh_1, b_1.reshape(1, D))
    p = _sc_agg(h, src_p, dst_p)
    h = _dense_lin(h, p, deg, W_self_2, W_neigh_2, b_2.reshape(1, D))
    return h[:N]
